# baseline (speedup 1.0000x reference)
import jax
import jax.numpy as jnp
from jax import lax
from jax.experimental import pallas as pl

_NB = 2048


def _body(eta_ref, xq_ref, xs_ref, wq_ref, bq_ref, ws_ref, bs_ref, out_ref):
    xq = xq_ref[...]
    xs = xs_ref[...]
    eta = eta_ref[...]
    e = wq_ref.shape[0]
    nb = xq.shape[1]

    dn = (((2,), (0,)), ((), ()))
    pq = lax.dot_general(wq_ref[...], xq, dn,
                         preferred_element_type=jnp.float32)
    ps = lax.dot_general(ws_ref[...], xs, dn,
                         preferred_element_type=jnp.float32)

    accq = pq[0]
    accs = ps[0]
    for i in range(1, e):
        m = eta == i
        accq = jnp.where(m, pq[i], accq)
        accs = jnp.where(m, ps[i], accs)

    oh = (lax.broadcasted_iota(jnp.int32, (e, nb), 0) == eta).astype(jnp.float32)
    dnb = (((0,), (0,)), ((), ()))
    biasq = lax.dot_general(bq_ref[...], oh, dnb,
                            preferred_element_type=jnp.float32)
    biass = lax.dot_general(bs_ref[...], oh, dnb,
                            preferred_element_type=jnp.float32)

    out_ref[0, :, :] = accq + biasq
    out_ref[1, :, :] = accs + biass


def kernel(QT, SLI, eta, W_QT, b_QT, W_SLI, b_SLI):
    nz, ny, nx = QT.shape
    e = W_QT.shape[0]
    n = ny * nx
    xq = QT.reshape(nz, n)
    xs = SLI.reshape(nz, n)
    eta2 = eta.reshape(1, n).astype(jnp.int32)

    out = pl.pallas_call(
        _body,
        grid=(n // _NB,),
        in_specs=[
            pl.BlockSpec((1, _NB), lambda i: (0, i)),
            pl.BlockSpec((nz, _NB), lambda i: (0, i)),
            pl.BlockSpec((nz, _NB), lambda i: (0, i)),
            pl.BlockSpec((e, nz, nz), lambda i: (0, 0, 0)),
            pl.BlockSpec((e, nz), lambda i: (0, 0)),
            pl.BlockSpec((e, nz, nz), lambda i: (0, 0, 0)),
            pl.BlockSpec((e, nz), lambda i: (0, 0)),
        ],
        out_specs=pl.BlockSpec((2, nz, _NB), lambda i: (0, 0, i)),
        out_shape=jax.ShapeDtypeStruct((2, nz, n), jnp.float32),
    )(eta2, xq, xs, W_QT, b_QT, W_SLI, b_SLI)
    return out.reshape(2, nz, ny, nx)


# speedup vs baseline: 1.0451x; 1.0451x over previous
import jax
import jax.numpy as jnp
from jax import lax
from jax.experimental import pallas as pl

_NB = 8192


def _body(eta_ref, xq_ref, xs_ref, wq_ref, bq_ref, ws_ref, bs_ref, out_ref):
    xq = xq_ref[...]
    xs = xs_ref[...]
    eta = eta_ref[...]
    e = wq_ref.shape[0]
    nb = xq.shape[1]

    dn = (((2,), (0,)), ((), ()))
    pq = lax.dot_general(wq_ref[...], xq, dn,
                         preferred_element_type=jnp.float32)
    ps = lax.dot_general(ws_ref[...], xs, dn,
                         preferred_element_type=jnp.float32)

    accq = pq[0]
    accs = ps[0]
    for i in range(1, e):
        m = eta == i
        accq = jnp.where(m, pq[i], accq)
        accs = jnp.where(m, ps[i], accs)

    oh = (lax.broadcasted_iota(jnp.int32, (e, nb), 0) == eta).astype(jnp.float32)
    dnb = (((0,), (0,)), ((), ()))
    biasq = lax.dot_general(bq_ref[...], oh, dnb,
                            preferred_element_type=jnp.float32)
    biass = lax.dot_general(bs_ref[...], oh, dnb,
                            preferred_element_type=jnp.float32)

    out_ref[0, :, :] = accq + biasq
    out_ref[1, :, :] = accs + biass


def kernel(QT, SLI, eta, W_QT, b_QT, W_SLI, b_SLI):
    nz, ny, nx = QT.shape
    e = W_QT.shape[0]
    n = ny * nx
    xq = QT.reshape(nz, n)
    xs = SLI.reshape(nz, n)
    eta2 = eta.reshape(1, n).astype(jnp.int32)

    out = pl.pallas_call(
        _body,
        grid=(n // _NB,),
        in_specs=[
            pl.BlockSpec((1, _NB), lambda i: (0, i)),
            pl.BlockSpec((nz, _NB), lambda i: (0, i)),
            pl.BlockSpec((nz, _NB), lambda i: (0, i)),
            pl.BlockSpec((e, nz, nz), lambda i: (0, 0, 0)),
            pl.BlockSpec((e, nz), lambda i: (0, 0)),
            pl.BlockSpec((e, nz, nz), lambda i: (0, 0, 0)),
            pl.BlockSpec((e, nz), lambda i: (0, 0)),
        ],
        out_specs=pl.BlockSpec((2, nz, _NB), lambda i: (0, 0, i)),
        out_shape=jax.ShapeDtypeStruct((2, nz, n), jnp.float32),
    )(eta2, xq, xs, W_QT, b_QT, W_SLI, b_SLI)
    return out.reshape(2, nz, ny, nx)


# speedup vs baseline: 2.0138x; 1.9268x over previous
import jax
import jax.numpy as jnp
from jax import lax
from jax.experimental import pallas as pl

_YB = 16


def _body(eta_ref, xq_ref, xs_ref, wq_ref, bq_ref, ws_ref, bs_ref, out_ref):
    nz, yb, nx = xq_ref.shape
    nb = yb * nx
    e = wq_ref.shape[0]
    xq = xq_ref[...].reshape(nz, nb)
    xs = xs_ref[...].reshape(nz, nb)
    eta = eta_ref[...].reshape(1, nb)

    dn = (((2,), (0,)), ((), ()))
    pq = lax.dot_general(wq_ref[...], xq, dn,
                         preferred_element_type=jnp.float32)
    ps = lax.dot_general(ws_ref[...], xs, dn,
                         preferred_element_type=jnp.float32)

    accq = pq[0]
    accs = ps[0]
    for i in range(1, e):
        m = eta == i
        accq = jnp.where(m, pq[i], accq)
        accs = jnp.where(m, ps[i], accs)

    oh = (lax.broadcasted_iota(jnp.int32, (e, nb), 0) == eta).astype(jnp.float32)
    dnb = (((0,), (0,)), ((), ()))
    biasq = lax.dot_general(bq_ref[...], oh, dnb,
                            preferred_element_type=jnp.float32)
    biass = lax.dot_general(bs_ref[...], oh, dnb,
                            preferred_element_type=jnp.float32)

    out_ref[0, :, :, :] = (accq + biasq).reshape(nz, yb, nx)
    out_ref[1, :, :, :] = (accs + biass).reshape(nz, yb, nx)


def kernel(QT, SLI, eta, W_QT, b_QT, W_SLI, b_SLI):
    nz, ny, nx = QT.shape
    e = W_QT.shape[0]
    eta32 = eta.astype(jnp.int32)

    out = pl.pallas_call(
        _body,
        grid=(ny // _YB,),
        in_specs=[
            pl.BlockSpec((_YB, nx), lambda i: (i, 0)),
            pl.BlockSpec((nz, _YB, nx), lambda i: (0, i, 0)),
            pl.BlockSpec((nz, _YB, nx), lambda i: (0, i, 0)),
            pl.BlockSpec((e, nz, nz), lambda i: (0, 0, 0)),
            pl.BlockSpec((e, nz), lambda i: (0, 0)),
            pl.BlockSpec((e, nz, nz), lambda i: (0, 0, 0)),
            pl.BlockSpec((e, nz), lambda i: (0, 0)),
        ],
        out_specs=pl.BlockSpec((2, nz, _YB, nx), lambda i: (0, 0, i, 0)),
        out_shape=jax.ShapeDtypeStruct((2, nz, ny, nx), jnp.float32),
    )(eta32, QT, SLI, W_QT, b_QT, W_SLI, b_SLI)
    return out


# speedup vs baseline: 2.4603x; 1.2217x over previous
import jax
import jax.numpy as jnp
from jax import lax
from jax.experimental import pallas as pl

_YB = 16
_ZP = 40


def _body(eta_ref, xq_ref, xs_ref, wq_ref, bq_ref, ws_ref, bs_ref, out_ref):
    nz, yb, nx = xq_ref.shape
    nb = yb * nx
    e = bq_ref.shape[1]
    xq = xq_ref[...].reshape(nz, nb)
    xs = xs_ref[...].reshape(nz, nb)
    eta = eta_ref[...].reshape(1, nb)

    pq = jnp.dot(wq_ref[...], xq, preferred_element_type=jnp.float32)
    ps = jnp.dot(ws_ref[...], xs, preferred_element_type=jnp.float32)

    accq = pq[0:nz]
    accs = ps[0:nz]
    for i in range(1, e):
        m = eta == i
        accq = jnp.where(m, pq[i * _ZP:i * _ZP + nz], accq)
        accs = jnp.where(m, ps[i * _ZP:i * _ZP + nz], accs)

    oh = (lax.broadcasted_iota(jnp.int32, (e, nb), 0) == eta).astype(jnp.float32)
    biasq = jnp.dot(bq_ref[...], oh, preferred_element_type=jnp.float32)
    biass = jnp.dot(bs_ref[...], oh, preferred_element_type=jnp.float32)

    out_ref[0, :, :, :] = (accq + biasq).reshape(nz, yb, nx)
    out_ref[1, :, :, :] = (accs + biass).reshape(nz, yb, nx)


def kernel(QT, SLI, eta, W_QT, b_QT, W_SLI, b_SLI):
    nz, ny, nx = QT.shape
    e = W_QT.shape[0]
    eta32 = eta.astype(jnp.int32)
    pad = ((0, 0), (0, _ZP - nz), (0, 0))
    wq = jnp.pad(W_QT, pad).reshape(e * _ZP, nz)
    ws = jnp.pad(W_SLI, pad).reshape(e * _ZP, nz)
    bqT = b_QT.T
    bsT = b_SLI.T

    out = pl.pallas_call(
        _body,
        grid=(ny // _YB,),
        in_specs=[
            pl.BlockSpec((_YB, nx), lambda i: (i, 0)),
            pl.BlockSpec((nz, _YB, nx), lambda i: (0, i, 0)),
            pl.BlockSpec((nz, _YB, nx), lambda i: (0, i, 0)),
            pl.BlockSpec((e * _ZP, nz), lambda i: (0, 0)),
            pl.BlockSpec((nz, e), lambda i: (0, 0)),
            pl.BlockSpec((e * _ZP, nz), lambda i: (0, 0)),
            pl.BlockSpec((nz, e), lambda i: (0, 0)),
        ],
        out_specs=pl.BlockSpec((2, nz, _YB, nx), lambda i: (0, 0, i, 0)),
        out_shape=jax.ShapeDtypeStruct((2, nz, ny, nx), jnp.float32),
    )(eta32, QT, SLI, wq, bqT, ws, bsT)
    return out


# speedup vs baseline: 2.7961x; 1.1365x over previous
import jax
import jax.numpy as jnp
from jax.experimental import pallas as pl

_YB = 32
_ZP = 40


def _body(eta_ref, xq_ref, xs_ref, wq_ref, ws_ref, out_ref):
    nz, yb, nx = xq_ref.shape
    nb = yb * nx
    e = wq_ref.shape[0] // _ZP
    ones = jnp.ones((1, nb), jnp.bfloat16)
    xq = jnp.concatenate(
        [xq_ref[...].astype(jnp.bfloat16).reshape(nz, nb), ones], axis=0)
    xs = jnp.concatenate(
        [xs_ref[...].astype(jnp.bfloat16).reshape(nz, nb), ones], axis=0)
    eta = eta_ref[...].reshape(1, nb)

    pq = jnp.dot(wq_ref[...], xq, preferred_element_type=jnp.float32)
    ps = jnp.dot(ws_ref[...], xs, preferred_element_type=jnp.float32)

    accq = pq[0:nz]
    accs = ps[0:nz]
    for i in range(1, e):
        m = eta == i
        accq = jnp.where(m, pq[i * _ZP:i * _ZP + nz], accq)
        accs = jnp.where(m, ps[i * _ZP:i * _ZP + nz], accs)

    out_ref[0, :, :, :] = accq.reshape(nz, yb, nx)
    out_ref[1, :, :, :] = accs.reshape(nz, yb, nx)


def kernel(QT, SLI, eta, W_QT, b_QT, W_SLI, b_SLI):
    nz, ny, nx = QT.shape
    e = W_QT.shape[0]
    eta32 = eta.astype(jnp.int32)
    wq = jnp.pad(jnp.concatenate([W_QT, b_QT[:, :, None]], axis=2),
                 ((0, 0), (0, _ZP - nz), (0, 0)))
    ws = jnp.pad(jnp.concatenate([W_SLI, b_SLI[:, :, None]], axis=2),
                 ((0, 0), (0, _ZP - nz), (0, 0)))
    wq = wq.reshape(e * _ZP, nz + 1).astype(jnp.bfloat16)
    ws = ws.reshape(e * _ZP, nz + 1).astype(jnp.bfloat16)

    out = pl.pallas_call(
        _body,
        grid=(ny // _YB,),
        in_specs=[
            pl.BlockSpec((_YB, nx), lambda i: (i, 0)),
            pl.BlockSpec((nz, _YB, nx), lambda i: (0, i, 0)),
            pl.BlockSpec((nz, _YB, nx), lambda i: (0, i, 0)),
            pl.BlockSpec((e * _ZP, nz + 1), lambda i: (0, 0)),
            pl.BlockSpec((e * _ZP, nz + 1), lambda i: (0, 0)),
        ],
        out_specs=pl.BlockSpec((2, nz, _YB, nx), lambda i: (0, 0, i, 0)),
        out_shape=jax.ShapeDtypeStruct((2, nz, ny, nx), jnp.float32),
    )(eta32, QT, SLI, wq, ws)
    return out
